# baseline (speedup 1.0000x reference)
import functools

import jax
import jax.numpy as jnp
from jax import lax
from jax.experimental import pallas as pl
from jax.experimental.pallas import tpu as pltpu
from jax.experimental.pallas import tpu_sc as plsc

_IDS = 8
_NB = 2


@functools.cache
def _build(batch: int, hist: int, vocab: int, dim: int):
    info = plsc.get_sparse_core_info()
    nw = info.num_cores * info.num_subcores
    assert batch % (nw * _IDS) == 0
    b_per_w = batch // nw
    nblocks = b_per_w // _IDS

    mesh = plsc.VectorSubcoreMesh(core_axis_name="c", subcore_axis_name="s")

    @functools.partial(
        pl.kernel,
        mesh=mesh,
        out_type=jax.ShapeDtypeStruct((batch, hist, dim), jnp.float32),
        scratch_types=[
            pltpu.VMEM((_NB, _IDS, hist), jnp.int32),
            pltpu.VMEM((_NB, hist, dim), jnp.float32),
            pltpu.SemaphoreType.DMA((_NB,)),
            pltpu.SemaphoreType.DMA((_NB,)),
            pltpu.SemaphoreType.DMA((_NB,)),
        ],
        compiler_params=pltpu.CompilerParams(use_tc_tiling_on_sc=False),
    )
    def gather_kernel(ids_hbm, table_hbm, out_hbm, idx_v, rows_v, isem, gsem, osem):
        wid = lax.axis_index("s") * info.num_cores + lax.axis_index("c")
        base = wid * b_per_w

        def ids_copy(blk, ib):
            return pltpu.make_async_copy(
                ids_hbm.at[pl.ds(base + blk * _IDS, _IDS)],
                idx_v.at[ib], isem.at[ib])

        def gathers(rr, ib, rb):
            cps = []
            off = 0
            while off < hist:
                n = min(128, hist - off)
                cps.append(pltpu.make_async_copy(
                    table_hbm.at[idx_v.at[ib, rr, pl.ds(off, n)]],
                    rows_v.at[rb, pl.ds(off, n)], gsem.at[rb]))
                off += n
            return cps

        def out_copy(r, rb):
            return pltpu.make_async_copy(
                rows_v.at[rb], out_hbm.at[base + r], osem.at[rb])

        ids_copy(0, 0).start()
        ids_copy(1, 1).start()
        ids_copy(0, 0).wait()
        for cp in gathers(0, 0, 0):
            cp.start()

        def row_body(r, carry):
            rb = lax.rem(r, _NB)
            nrb = lax.rem(r + 1, _NB)
            blk = r // _IDS
            ib = lax.rem(blk, _NB)
            @pl.when(r + 1 < b_per_w)
            def _():
                nblk = (r + 1) // _IDS
                nib = lax.rem(nblk, _NB)
                @pl.when(lax.rem(r + 1, _IDS) == 0)
                def _():
                    ids_copy(nblk, nib).wait()
                @pl.when(r >= 1)
                def _():
                    out_copy(r - 1, nrb).wait()
                for cp in gathers(lax.rem(r + 1, _IDS), nib, nrb):
                    cp.start()
            for cp in gathers(lax.rem(r, _IDS), ib, rb):
                cp.wait()
            out_copy(r, rb).start()
            @pl.when((lax.rem(r, _IDS) == _IDS - 1) & (blk + 2 < nblocks))
            def _():
                ids_copy(blk + 2, ib).start()
            return carry

        lax.fori_loop(0, b_per_w, row_body, 0)
        out_copy(b_per_w - 2, lax.rem(b_per_w - 2, _NB)).wait()
        out_copy(b_per_w - 1, lax.rem(b_per_w - 1, _NB)).wait()

    return gather_kernel


def kernel(input_ids, table):
    batch, hist = input_ids.shape
    vocab, dim = table.shape
    ids = input_ids.astype(jnp.int32)
    return _build(batch, hist, vocab, dim)(ids, table)
